# baseline (speedup 1.0000x reference)
import jax
import jax.numpy as jnp
from jax.experimental import pallas as pl
from jax.experimental.pallas import tpu as pltpu

_BN = 512


def _mlp_center_kernel(x_ref, w1_ref, b1_ref, w2_ref, out_ref, acc_ref):
    nb = pl.program_id(1)
    num_nb = pl.num_programs(1)

    x = x_ref[0].astype(jnp.bfloat16)
    h = jnp.dot(x, w1_ref[...], preferred_element_type=jnp.float32)
    h = jnp.maximum(h + b1_ref[...], 0.0)
    logits = jnp.sum(h * w2_ref[...], axis=1)

    s = jnp.sum(logits)

    @pl.when(nb == 0)
    def _init():
        acc_ref[0, 0] = s

    @pl.when(nb != 0)
    def _accum():
        acc_ref[0, 0] += s

    out_ref[0, 0, pl.ds(nb * _BN, _BN)] = logits

    @pl.when(nb == num_nb - 1)
    def _center():
        mean = acc_ref[0, 0] / out_ref.shape[2]
        out_ref[0, 0, :] = out_ref[0, 0, :] - mean


def kernel(x, W1, b1, W2, b2):
    del b2
    B, N, D = x.shape
    w1 = W1.astype(jnp.bfloat16)
    b1r = b1.reshape(1, D)
    w2r = W2.reshape(1, D)

    out = pl.pallas_call(
        _mlp_center_kernel,
        grid=(B, N // _BN),
        in_specs=[
            pl.BlockSpec((1, _BN, D), lambda b, nb: (b, nb, 0)),
            pl.BlockSpec((D, D), lambda b, nb: (0, 0)),
            pl.BlockSpec((1, D), lambda b, nb: (0, 0)),
            pl.BlockSpec((1, D), lambda b, nb: (0, 0)),
        ],
        out_specs=pl.BlockSpec((1, 1, N), lambda b, nb: (b, 0, 0)),
        out_shape=jax.ShapeDtypeStruct((B, 1, N), jnp.float32),
        scratch_shapes=[pltpu.SMEM((1, 1), jnp.float32)],
        compiler_params=pltpu.CompilerParams(
            dimension_semantics=("parallel", "arbitrary"),
        ),
    )(x, w1, b1r, w2r)
    return out.reshape(B, N)
